# baseline (speedup 1.0000x reference)
import functools
import math

import jax
import jax.numpy as jnp
from jax import lax
from jax.experimental import pallas as pl
from jax.experimental.pallas import tpu as pltpu
from jax.experimental.pallas import tpu_sc as plsc

D = 64
SCALE_F = math.sqrt(D)
NC = 2
NS = 16
NW = NC * NS
L = 16

B_TOTAL = 4096 * 200
B_PER_W = B_TOTAL // NW
CH = 128
NCHUNK = B_PER_W // CH
NB = 2
ROW_UNROLL = 16


def _body(table_hbm, x_hbm, out_hbm, idx_v, in_bufs, out_bufs, gsems, ssems):
    wid = lax.axis_index("s") * NC + lax.axis_index("c")
    base = wid * B_PER_W

    pltpu.sync_copy(x_hbm.at[pl.ds(base, B_PER_W)], idx_v)

    def gather_start(g, b):
        pltpu.async_copy(
            table_hbm.at[idx_v.at[pl.ds(g * CH, CH)]], in_bufs[b], gsems[b])

    def gather_wait(b):
        pltpu.make_async_copy(
            table_hbm.at[idx_v.at[pl.ds(0, CH)]], in_bufs[b], gsems[b]).wait()

    def scatter_start(g, b):
        pltpu.async_copy(
            out_bufs[b], out_hbm.at[pl.ds(base + g * CH, CH)], ssems[b])

    def scatter_wait(b):
        pltpu.make_async_copy(
            out_bufs[b], out_hbm.at[pl.ds(0, CH)], ssems[b]).wait()

    for b in range(NB):
        gather_start(b, b)

    @pl.loop(0, NCHUNK, step=NB)
    def _(g0):
        for b in range(NB):
            g = g0 + b
            gather_wait(b)

            @pl.when(g0 > 0)
            def _():
                scatter_wait(b)

            ib = in_bufs[b]
            ob = out_bufs[b]
            goff = g * CH

            @pl.loop(0, CH, step=ROW_UNROLL)
            def _(r0):
                ivv = idx_v[pl.ds(goff + r0, L)]
                sv = jnp.where(ivv == 0, jnp.float32(0.0),
                               jnp.float32(SCALE_F))
                for dr in range(ROW_UNROLL):
                    r = r0 + dr
                    s = sv[dr]
                    for q in range(D // L):
                        ob[r, pl.ds(q * L, L)] = ib[r, pl.ds(q * L, L)] * s

            scatter_start(g, b)

            @pl.when(g + NB < NCHUNK)
            def _():
                gather_start(g + NB, b)

    for b in range(NB):
        scatter_wait(b)


@jax.jit
def _run(x_flat, table):
    mesh = plsc.VectorSubcoreMesh(core_axis_name="c", subcore_axis_name="s")
    f = pl.kernel(
        _body,
        out_type=jax.ShapeDtypeStruct((B_TOTAL, D), jnp.float32),
        mesh=mesh,
        scratch_types=[
            pltpu.VMEM((B_PER_W,), jnp.int32),
            [pltpu.VMEM((CH, D), jnp.float32) for _ in range(NB)],
            [pltpu.VMEM((CH, D), jnp.float32) for _ in range(NB)],
            [pltpu.SemaphoreType.DMA for _ in range(NB)],
            [pltpu.SemaphoreType.DMA for _ in range(NB)],
        ],
        compiler_params=pltpu.CompilerParams(use_tc_tiling_on_sc=False),
    )
    return f(table, x_flat)


def kernel(x, table):
    out = _run(x.reshape(-1), table)
    return out.reshape(x.shape[0], x.shape[1], D)
